# baseline (speedup 1.0000x reference)
import functools

import jax
import jax.numpy as jnp
from jax import lax
from jax.experimental import pallas as pl
from jax.experimental.pallas import tpu as pltpu
from jax.experimental.pallas import tpu_sc as plsc

B, H, W = 128, 512, 512
OUT_H, OUT_W = 480, 480
NC, NS = 2, 16
NW = NC * NS
IMGS_PER_W = B // NW
CHUNKS = 10
CHUNK_ROWS = OUT_H // CHUNKS
L = 16
ROW_UNROLL = 4


def kernel(x):
    mesh = plsc.VectorSubcoreMesh(core_axis_name="c", subcore_axis_name="s")

    @functools.partial(
        pl.kernel,
        mesh=mesh,
        out_type=jax.ShapeDtypeStruct((B, OUT_H, OUT_W), jnp.float32),
        scratch_types=[
            pltpu.VMEM((CHUNK_ROWS, W), jnp.float32),
            pltpu.VMEM((CHUNK_ROWS, W), jnp.float32),
            pltpu.VMEM((CHUNK_ROWS, OUT_W), jnp.float32),
            pltpu.VMEM((CHUNK_ROWS, OUT_W), jnp.float32),
            pltpu.VMEM((CHUNK_ROWS,), jnp.int32),
            pltpu.VMEM((CHUNK_ROWS,), jnp.int32),
            pltpu.SemaphoreType.DMA,
            pltpu.SemaphoreType.DMA,
            pltpu.SemaphoreType.DMA,
            pltpu.SemaphoreType.DMA,
        ],
        compiler_params=pltpu.CompilerParams(needs_layout_passes=False),
    )
    def body(x_hbm, out_hbm, buf0, buf1, obuf0, obuf1, idx0, idx1,
             isem0, isem1, osem0, osem1):
        wid = lax.axis_index("s") * NC + lax.axis_index("c")
        base = wid * IMGS_PER_W
        bufs = (buf0, buf1)
        obufs = (obuf0, obuf1)
        idxs = (idx0, idx1)
        isems = (isem0, isem1)
        osems = (osem0, osem1)
        lane = lax.iota(jnp.int32, L)
        n_items = IMGS_PER_W * CHUNKS

        def item(t):
            img = base + (t // CHUNKS)
            off = 2 - lax.rem(img, 3)
            row0 = (t % CHUNKS) * CHUNK_ROWS
            return img, off, row0

        def start_in(t, b):
            img, off, row0 = item(t)
            r0 = img * H + off + row0
            for v in range(CHUNK_ROWS // L):
                idxs[b][pl.ds(v * L, L)] = lane + (r0 + v * L)
            pltpu.async_copy(x_hbm.at[idxs[b]], bufs[b], isems[b])

        def wait_in(b):
            pltpu.make_async_copy(
                x_hbm.at[pl.ds(0, CHUNK_ROWS), :], bufs[b], isems[b]).wait()

        def start_out(t, b):
            img, off, row0 = item(t)
            pltpu.async_copy(obufs[b],
                             out_hbm.at[img, pl.ds(row0, CHUNK_ROWS), :],
                             osems[b])

        def wait_out(b):
            pltpu.make_async_copy(
                obufs[b], out_hbm.at[0, pl.ds(0, CHUNK_ROWS), :],
                osems[b]).wait()

        def shift(buf, obuf, off):
            def row_body(rb, _):
                for u in range(ROW_UNROLL):
                    r = rb * ROW_UNROLL + u
                    row_idx = jnp.full((L,), r, jnp.int32)
                    for c in range(OUT_W // L):
                        col_idx = lane + (off + c * L)
                        v = plsc.load_gather(buf, [row_idx, col_idx])
                        obuf[r, pl.ds(c * L, L)] = v
                return 0

            lax.fori_loop(0, CHUNK_ROWS // ROW_UNROLL, row_body, 0)

        start_in(0, 0)
        start_in(1, 1)

        def outer_body(t2, _):
            for b in range(2):
                t = t2 * 2 + b
                wait_in(b)
                pl.when(t2 >= 1)(lambda: wait_out(b))
                img, off, row0 = item(t)
                shift(bufs[b], obufs[b], off)
                start_out(t, b)
                pl.when(t2 < n_items // 2 - 1)(lambda: start_in(t + 2, b))
            return 0

        lax.fori_loop(0, n_items // 2, outer_body, 0)
        wait_out(0)
        wait_out(1)

    return body(x.reshape(B * H, W))


# speedup vs baseline: 1.5160x; 1.5160x over previous
import functools

import jax
import jax.numpy as jnp
from jax import lax
from jax.experimental import pallas as pl
from jax.experimental.pallas import tpu as pltpu
from jax.experimental.pallas import tpu_sc as plsc

B, H, W = 128, 512, 512
OUT_H, OUT_W = 480, 480
NC, NS = 2, 16
NW = NC * NS
IMGS_PER_W = B // NW
CHUNKS = 10
CHUNK_ROWS = OUT_H // CHUNKS
L = 16
ROW_UNROLL = 4


def kernel(x):
    mesh = plsc.VectorSubcoreMesh(core_axis_name="c", subcore_axis_name="s")

    @functools.partial(
        pl.kernel,
        mesh=mesh,
        out_type=jax.ShapeDtypeStruct((B, OUT_H, OUT_W), jnp.float32),
        scratch_types=[
            pltpu.VMEM((CHUNK_ROWS, W), jnp.float32),
            pltpu.VMEM((CHUNK_ROWS, W), jnp.float32),
            pltpu.VMEM((CHUNK_ROWS, OUT_W), jnp.float32),
            pltpu.VMEM((CHUNK_ROWS, OUT_W), jnp.float32),
            pltpu.VMEM((CHUNK_ROWS,), jnp.int32),
            pltpu.VMEM((CHUNK_ROWS,), jnp.int32),
            pltpu.SemaphoreType.DMA,
            pltpu.SemaphoreType.DMA,
            pltpu.SemaphoreType.DMA,
            pltpu.SemaphoreType.DMA,
        ],
        compiler_params=pltpu.CompilerParams(needs_layout_passes=False),
    )
    def body(x_hbm, out_hbm, buf0, buf1, obuf0, obuf1, idx0, idx1,
             isem0, isem1, osem0, osem1):
        wid = lax.axis_index("s") * NC + lax.axis_index("c")
        base = wid * IMGS_PER_W
        bufs = (buf0, buf1)
        obufs = (obuf0, obuf1)
        idxs = (idx0, idx1)
        isems = (isem0, isem1)
        osems = (osem0, osem1)
        lane = lax.iota(jnp.int32, L)
        n_items = IMGS_PER_W * CHUNKS

        def item(t):
            img = base + (t // CHUNKS)
            off = 2 - lax.rem(img, 3)
            row0 = (t % CHUNKS) * CHUNK_ROWS
            return img, off, row0

        def start_in(t, b):
            img, off, row0 = item(t)
            r0 = img * H + off + row0
            for v in range(CHUNK_ROWS // L):
                idxs[b][pl.ds(v * L, L)] = lane + (r0 + v * L)
            pltpu.async_copy(x_hbm.at[idxs[b]], bufs[b], isems[b])

        def wait_in(b):
            pltpu.make_async_copy(
                x_hbm.at[pl.ds(0, CHUNK_ROWS), :], bufs[b], isems[b]).wait()

        def start_out(t, b):
            img, off, row0 = item(t)
            pltpu.async_copy(obufs[b],
                             out_hbm.at[img, pl.ds(row0, CHUNK_ROWS), :],
                             osems[b])

        def wait_out(b):
            pltpu.make_async_copy(
                obufs[b], out_hbm.at[0, pl.ds(0, CHUNK_ROWS), :],
                osems[b]).wait()

        def shift(buf, obuf, off):
            @plsc.parallel_loop(0, CHUNK_ROWS, 1, unroll=ROW_UNROLL)
            def row_body(r):
                row_idx = jnp.full((L,), r, jnp.int32)
                for c in range(OUT_W // L):
                    col_idx = lane + (off + c * L)
                    v = plsc.load_gather(buf, [row_idx, col_idx])
                    obuf[r, pl.ds(c * L, L)] = v

        start_in(0, 0)
        start_in(1, 1)

        def outer_body(t2, _):
            for b in range(2):
                t = t2 * 2 + b
                wait_in(b)
                pl.when(t2 >= 1)(lambda: wait_out(b))
                img, off, row0 = item(t)
                shift(bufs[b], obufs[b], off)
                start_out(t, b)
                pl.when(t2 < n_items // 2 - 1)(lambda: start_in(t + 2, b))
            return 0

        lax.fori_loop(0, n_items // 2, outer_body, 0)
        wait_out(0)
        wait_out(1)

    return body(x.reshape(B * H, W))
